# baseline (speedup 1.0000x reference)
import functools

import jax
import jax.numpy as jnp
from jax import lax
from jax.experimental import pallas as pl
from jax.experimental.pallas import tpu as pltpu
from jax.experimental.pallas import tpu_sc as plsc

NU = 10000
NI = 10000
E = 160000
E2 = E // 2
D = 256
H = 4
DH = 64
DEA = 16
HID = 128

NCORE = 2
NSUB = 16
NW = NCORE * NSUB

C = 2000
G = C // 16

TBASE = (0, 1, 0, 1)

NB = 1000
NBLK = NU // NB


_sc_mesh = plsc.VectorSubcoreMesh(core_axis_name="c", subcore_axis_name="s")
_sc_params = pltpu.CompilerParams(
    needs_layout_passes=False, use_tc_tiling_on_sc=False
)


def _sc_body(xtab,
             src_f, dst_f, src_b, dst_b, src_u, dst_u, src_s, dst_s,
             ea_f, ea_b, ea_u, ea_s, z2, z1, aggx, aggeP, degp,
             acc2, acc1, sidx0, sidx1, didx0, didx1, rows0, rows1, onesb,
             semG0, semG1, semD0, semD1, semS0, semS1):
    wid = lax.axis_index("s") * NCORE + lax.axis_index("c")
    sid = lax.axis_index("s")
    a2b = sid * 10000
    a1b = (sid - 8) * 10000
    w8 = wid * 8

    srcs = (src_f, src_b, src_u, src_s)
    dsts = (dst_f, dst_b, dst_u, dst_s)
    eas = (ea_f, ea_b, ea_u, ea_s)

    def run_pipe(issue, dload, nch):
        def wait_gather(b):
            sidxb, rowsb, semg = (sidx0, rows0, semG0) if b == 0 else \
                                 (sidx1, rows1, semG1)
            issue(0, sidxb, rowsb, semg, wait_only=True)

        def wait_didx(b):
            didxb, semd = (didx0, semD0) if b == 0 else (didx1, semD1)
            dload(0, didxb, semd, wait_only=True)

        def scat(b):
            rowsb, didxb, sems = (rows0, didx0, semS0) if b == 0 else \
                                 (rows1, didx1, semS1)
            return pltpu.async_copy(
                rowsb, acc2.at[pl.ds(a2b, 10000)].at[didxb], sems, add=True)

        def wait_scat(b):
            rowsb, didxb, sems = (rows0, didx0, semS0) if b == 0 else \
                                 (rows1, didx1, semS1)
            pltpu.make_async_copy(
                rowsb, acc2.at[pl.ds(a2b, 10000)].at[didxb], sems).wait()

        issue(0, sidx0, rows0, semG0)
        dload(0, didx0, semD0)
        issue(1, sidx1, rows1, semG1)
        dload(1, didx1, semD1)

        def body(i, _):
            wait_gather(0)
            wait_didx(0)
            scat(0)
            wait_gather(1)
            wait_didx(1)
            scat(1)
            wait_scat(0)
            issue(2 * i + 2, sidx0, rows0, semG0)
            dload(2 * i + 2, didx0, semD0)
            wait_scat(1)
            issue(2 * i + 3, sidx1, rows1, semG1)
            dload(2 * i + 3, didx1, semD1)
            return _

        lax.fori_loop(0, nch // 2 - 1, body, 0)
        wait_gather(0)
        wait_didx(0)
        scat(0)
        wait_gather(1)
        wait_didx(1)
        scat(1)
        wait_scat(0)
        wait_scat(1)

    for r in range(4):
        sb = TBASE[r] * (32 * 10000) + wid * 10000
        src_r = srcs[r]
        dst_r = dsts[r]
        pltpu.sync_copy(z2, acc2.at[pl.ds(a2b, 10000), :])

        def issue(c, sidxb, rowsb, sem, wait_only=False, sb=sb, src_r=src_r):
            if wait_only:
                pltpu.make_async_copy(
                    xtab.at[pl.ds(sb, 10000)].at[sidxb], rowsb, sem).wait()
                return
            pltpu.sync_copy(src_r.at[pl.ds(c * C, C)], sidxb)
            pltpu.async_copy(xtab.at[pl.ds(sb, 10000)].at[sidxb], rowsb, sem)

        def dload(c, didxb, sem, wait_only=False, dst_r=dst_r):
            if wait_only:
                pltpu.make_async_copy(
                    dst_r.at[pl.ds(0, C)], didxb, sem).wait()
                return
            pltpu.async_copy(dst_r.at[pl.ds(c * C, C)], didxb, sem)

        run_pipe(issue, dload, E // C)
        pltpu.sync_copy(acc2.at[pl.ds(a2b, 10000), :],
                        aggx.at[pl.ds(r * 10000, 10000), pl.ds(w8, 8)])

    for r in range(4):
        @pl.when(wid // 4 == r)
        def _(r=r):
            h = (wid // 2) % 2
            half = wid % 2
            eoff0 = half * E2
            ea_r = eas[r]
            dst_r = dsts[r]
            pltpu.sync_copy(z2, acc2.at[pl.ds(a2b, 10000), :])

            def issue(c, sidxb, rowsb, sem, wait_only=False):
                ref = ea_r.at[pl.ds(eoff0 + c * C, C), pl.ds(h * 8, 8)]
                if wait_only:
                    pltpu.make_async_copy(ref, rowsb, sem).wait()
                else:
                    pltpu.async_copy(ref, rowsb, sem)

            def dload(c, didxb, sem, wait_only=False):
                ref = dst_r.at[pl.ds(eoff0 + c * C, C)]
                if wait_only:
                    pltpu.make_async_copy(ref, didxb, sem).wait()
                else:
                    pltpu.async_copy(ref, didxb, sem)

            run_pipe(issue, dload, E2 // C)
            pltpu.sync_copy(
                acc2.at[pl.ds(a2b, 10000), :],
                aggeP.at[pl.ds(half * 40000 + r * 10000, 10000),
                         pl.ds(h * 8, 8)])

    for r in range(4):
        @pl.when((wid >= 16) & (wid < 24) & ((wid - 16) // 2 == r))
        def _(r=r):
            half = wid % 2
            doff0 = half * E2
            dst_r = dsts[r]
            pltpu.sync_copy(z1, acc1.at[pl.ds(a1b, 10000)])

            def fill(g, _):
                onesb[pl.ds(g * 16, 16)] = jnp.ones((16,), jnp.float32)
                return _

            lax.fori_loop(0, G, fill, 0, unroll=8)

            def chunk_body3(c, _):
                pltpu.sync_copy(dst_r.at[pl.ds(doff0 + c * C, C)], didx0)
                pltpu.sync_copy(onesb, acc1.at[pl.ds(a1b, 10000)].at[didx0],
                                add=True)
                return _

            lax.fori_loop(0, E2 // C, chunk_body3, 0)
            pltpu.sync_copy(acc1.at[pl.ds(a1b, 10000)],
                            degp.at[pl.ds((r * 2 + half) * 10000, 10000)])


_sc_aggregate = functools.partial(
    pl.kernel,
    mesh=_sc_mesh,
    compiler_params=_sc_params,
    out_type=[
        jax.ShapeDtypeStruct((4 * 10000, 256), jnp.float32),
        jax.ShapeDtypeStruct((2 * 4 * 10000, 16), jnp.float32),
        jax.ShapeDtypeStruct((8 * 10000,), jnp.float32),
    ],
    scratch_types=[
        pltpu.VMEM_SHARED((NSUB * 10000, 8), jnp.float32),
        pltpu.VMEM_SHARED((4 * 10000,), jnp.float32),
        pltpu.VMEM((C,), jnp.int32),
        pltpu.VMEM((C,), jnp.int32),
        pltpu.VMEM((C,), jnp.int32),
        pltpu.VMEM((C,), jnp.int32),
        pltpu.VMEM((C, 8), jnp.float32),
        pltpu.VMEM((C, 8), jnp.float32),
        pltpu.VMEM((C,), jnp.float32),
        pltpu.SemaphoreType.DMA,
        pltpu.SemaphoreType.DMA,
        pltpu.SemaphoreType.DMA,
        pltpu.SemaphoreType.DMA,
        pltpu.SemaphoreType.DMA,
        pltpu.SemaphoreType.DMA,
    ],
)(_sc_body)


def _p1_body(a0, e0, d0, a1, e1, d1, W0, We0, W1r, We1r, W1, b1, w2r,
             z0o, z1o, s0o, s1o, acc):
    i = pl.program_id(0)

    @pl.when(i == 0)
    def _():
        acc[0] = 0.0
        acc[1] = 0.0

    rd0 = 1.0 / jnp.maximum(d0[...], 1.0)
    rd1 = 1.0 / jnp.maximum(d1[...], 1.0)
    z0 = (jnp.dot(a0[...], W0[...], preferred_element_type=jnp.float32)
          + jnp.dot(e0[...], We0[...], preferred_element_type=jnp.float32)) * rd0
    z1 = (jnp.dot(a1[...], W1r[...], preferred_element_type=jnp.float32)
          + jnp.dot(e1[...], We1r[...], preferred_element_type=jnp.float32)) * rd1
    h0 = jnp.tanh(jnp.dot(z0, W1[...], preferred_element_type=jnp.float32) + b1[...])
    h1 = jnp.tanh(jnp.dot(z1, W1[...], preferred_element_type=jnp.float32) + b1[...])
    acc[0] += jnp.sum(h0 * w2r[...])
    acc[1] += jnp.sum(h1 * w2r[...])
    z0o[...] = z0
    z1o[...] = z1

    @pl.when(i == NBLK - 1)
    def _():
        s0o[...] = jnp.full((1, 128), acc[0], jnp.float32)
        s1o[...] = jnp.full((1, 128), acc[1], jnp.float32)


def _p2_body(z0, z1, s0, s1, o):
    t0 = s0[0, 0] * (1.0 / NU)
    t1 = s1[0, 0] * (1.0 / NU)
    m = jnp.maximum(t0, t1)
    e0 = jnp.exp(t0 - m)
    e1 = jnp.exp(t1 - m)
    b0 = e0 / (e0 + e1)
    b1 = e1 / (e0 + e1)
    o[...] = b0 * z0[...] + b1 * z1[...]


def _dense_pair(a0, e0, d0, a1, e1, d1, W0, We0, W1r, We1r, W1, b1, w2):
    row = lambda i: (i, 0)
    const = lambda i: (0, 0)
    z0, z1, s0, s1 = pl.pallas_call(
        _p1_body,
        grid=(NBLK,),
        in_specs=[
            pl.BlockSpec((NB, D), row),
            pl.BlockSpec((NB, DEA), row),
            pl.BlockSpec((NB, 1), row),
            pl.BlockSpec((NB, D), row),
            pl.BlockSpec((NB, DEA), row),
            pl.BlockSpec((NB, 1), row),
            pl.BlockSpec((D, D), const),
            pl.BlockSpec((DEA, D), const),
            pl.BlockSpec((D, D), const),
            pl.BlockSpec((DEA, D), const),
            pl.BlockSpec((D, HID), const),
            pl.BlockSpec((1, HID), const),
            pl.BlockSpec((1, HID), const),
        ],
        out_specs=[
            pl.BlockSpec((NB, D), row),
            pl.BlockSpec((NB, D), row),
            pl.BlockSpec((1, 128), const),
            pl.BlockSpec((1, 128), const),
        ],
        out_shape=[
            jax.ShapeDtypeStruct((NU, D), jnp.float32),
            jax.ShapeDtypeStruct((NU, D), jnp.float32),
            jax.ShapeDtypeStruct((1, 128), jnp.float32),
            jax.ShapeDtypeStruct((1, 128), jnp.float32),
        ],
        scratch_shapes=[pltpu.SMEM((2,), jnp.float32)],
    )(a0, e0, d0.reshape(NU, 1), a1, e1, d1.reshape(NU, 1),
      W0, We0, W1r, We1r, W1, b1.reshape(1, HID), w2.reshape(1, HID))

    out = pl.pallas_call(
        _p2_body,
        grid=(NBLK,),
        in_specs=[
            pl.BlockSpec((NB, D), row),
            pl.BlockSpec((NB, D), row),
            pl.BlockSpec((1, 128), const),
            pl.BlockSpec((1, 128), const),
        ],
        out_specs=pl.BlockSpec((NB, D), row),
        out_shape=jax.ShapeDtypeStruct((NU, D), jnp.float32),
    )(z0, z1, s0, s1)
    return out.reshape(NU, H, DH)


def kernel(x_user, x_item, ei_follows, ei_boughtby, ei_buys, ei_similar,
           ea_follows, ea_boughtby, ea_buys, ea_similar,
           W_follows, We_follows, W_boughtby, We_boughtby,
           W_buys, We_buys, W_similar, We_similar,
           W1_u, b1_u, w2_u, W1_i, b1_i, w2_i):
    xu3 = x_user.reshape(NU, 32, 8).transpose(1, 0, 2).reshape(-1, 8)
    xi3 = x_item.reshape(NI, 32, 8).transpose(1, 0, 2).reshape(-1, 8)
    xtab = jnp.concatenate([xu3, xi3], axis=0)

    z2 = jnp.zeros((10000, 8), jnp.float32)
    z1 = jnp.zeros((10000,), jnp.float32)

    aggx, aggeP, degp = _sc_aggregate(
        xtab,
        ei_follows[0], ei_follows[1], ei_boughtby[0], ei_boughtby[1],
        ei_buys[0], ei_buys[1], ei_similar[0], ei_similar[1],
        ea_follows, ea_boughtby, ea_buys, ea_similar, z2, z1)

    aggx4 = aggx.reshape(4, NU, D)
    agge4 = aggeP.reshape(2, 4, NU, DEA).sum(axis=0)
    deg4 = degp.reshape(4, 2, NU).sum(axis=1)

    out_user = _dense_pair(aggx4[0], agge4[0], deg4[0],
                           aggx4[1], agge4[1], deg4[1],
                           W_follows, We_follows, W_boughtby, We_boughtby,
                           W1_u, b1_u, w2_u)
    out_item = _dense_pair(aggx4[2], agge4[2], deg4[2],
                           aggx4[3], agge4[3], deg4[3],
                           W_buys, We_buys, W_similar, We_similar,
                           W1_i, b1_i, w2_i)
    return (out_user, out_item)


# speedup vs baseline: 1.0350x; 1.0350x over previous
import functools

import jax
import jax.numpy as jnp
from jax import lax
from jax.experimental import pallas as pl
from jax.experimental.pallas import tpu as pltpu
from jax.experimental.pallas import tpu_sc as plsc

NU = 10000
NI = 10000
E = 160000
E2 = E // 2
D = 256
H = 4
DH = 64
DEA = 16
HID = 128

NCORE = 2
NSUB = 16
NW = NCORE * NSUB

C = 2000
G = C // 16

TBASE = (0, 1, 0, 1)

NB = 1000
NBLK = NU // NB


_sc_mesh = plsc.VectorSubcoreMesh(core_axis_name="c", subcore_axis_name="s")
_sc_params = pltpu.CompilerParams(
    needs_layout_passes=False, use_tc_tiling_on_sc=False
)


def _sc_body(xtab,
             src_f, dst_f, src_b, dst_b, src_u, dst_u, src_s, dst_s,
             ea_f, ea_b, ea_u, ea_s, z2, z1, aggx, aggeP, degp,
             acc2, acc1, sidx0, sidx1, didx0, didx1, rows0, rows1, onesb,
             semG0, semG1, semD0, semD1, semS0, semS1):
    wid = lax.axis_index("s") * NCORE + lax.axis_index("c")
    sid = lax.axis_index("s")
    a2b = sid * 10000
    a1b = (sid - 8) * 10000
    w8 = wid * 8

    srcs = (src_f, src_b, src_u, src_s)
    dsts = (dst_f, dst_b, dst_u, dst_s)
    eas = (ea_f, ea_b, ea_u, ea_s)

    def run_pipe(issue, dload, nch):
        def wait_gather(b):
            sidxb, rowsb, semg = (sidx0, rows0, semG0) if b == 0 else \
                                 (sidx1, rows1, semG1)
            issue(0, sidxb, rowsb, semg, wait_only=True)

        def scat(rowsb, didxb):
            pltpu.sync_copy(rowsb, acc2.at[pl.ds(a2b, 10000)].at[didxb],
                            add=True)

        issue(0, sidx0, rows0, semG0)

        def body(i, _):
            issue(2 * i + 1, sidx1, rows1, semG1)
            dload(2 * i, didx0, semD0)
            dload(0, didx0, semD0, wait_only=True)
            wait_gather(0)
            scat(rows0, didx0)
            issue(2 * i + 2, sidx0, rows0, semG0)
            dload(2 * i + 1, didx1, semD1)
            dload(0, didx1, semD1, wait_only=True)
            wait_gather(1)
            scat(rows1, didx1)
            return _

        lax.fori_loop(0, nch // 2 - 1, body, 0)
        issue(nch - 1, sidx1, rows1, semG1)
        dload(nch - 2, didx0, semD0)
        dload(0, didx0, semD0, wait_only=True)
        wait_gather(0)
        scat(rows0, didx0)
        dload(nch - 1, didx1, semD1)
        dload(0, didx1, semD1, wait_only=True)
        wait_gather(1)
        scat(rows1, didx1)

    for r in range(4):
        sb = TBASE[r] * (32 * 10000) + wid * 10000
        src_r = srcs[r]
        dst_r = dsts[r]
        pltpu.sync_copy(z2, acc2.at[pl.ds(a2b, 10000), :])

        def issue(c, sidxb, rowsb, sem, wait_only=False, sb=sb, src_r=src_r):
            if wait_only:
                pltpu.make_async_copy(
                    xtab.at[pl.ds(sb, 10000)].at[sidxb], rowsb, sem).wait()
                return
            pltpu.sync_copy(src_r.at[pl.ds(c * C, C)], sidxb)
            pltpu.async_copy(xtab.at[pl.ds(sb, 10000)].at[sidxb], rowsb, sem)

        def dload(c, didxb, sem, wait_only=False, dst_r=dst_r):
            if wait_only:
                pltpu.make_async_copy(
                    dst_r.at[pl.ds(0, C)], didxb, sem).wait()
                return
            pltpu.async_copy(dst_r.at[pl.ds(c * C, C)], didxb, sem)

        run_pipe(issue, dload, E // C)
        pltpu.sync_copy(acc2.at[pl.ds(a2b, 10000), :],
                        aggx.at[pl.ds(r * 10000, 10000), pl.ds(w8, 8)])

    for r in range(4):
        @pl.when(wid // 4 == r)
        def _(r=r):
            h = (wid // 2) % 2
            half = wid % 2
            eoff0 = half * E2
            ea_r = eas[r]
            dst_r = dsts[r]
            pltpu.sync_copy(z2, acc2.at[pl.ds(a2b, 10000), :])

            def issue(c, sidxb, rowsb, sem, wait_only=False):
                ref = ea_r.at[pl.ds(eoff0 + c * C, C), pl.ds(h * 8, 8)]
                if wait_only:
                    pltpu.make_async_copy(ref, rowsb, sem).wait()
                else:
                    pltpu.async_copy(ref, rowsb, sem)

            def dload(c, didxb, sem, wait_only=False):
                ref = dst_r.at[pl.ds(eoff0 + c * C, C)]
                if wait_only:
                    pltpu.make_async_copy(ref, didxb, sem).wait()
                else:
                    pltpu.async_copy(ref, didxb, sem)

            run_pipe(issue, dload, E2 // C)
            pltpu.sync_copy(
                acc2.at[pl.ds(a2b, 10000), :],
                aggeP.at[pl.ds(half * 40000 + r * 10000, 10000),
                         pl.ds(h * 8, 8)])

    for r in range(4):
        @pl.when((wid >= 16) & (wid < 24) & ((wid - 16) // 2 == r))
        def _(r=r):
            half = wid % 2
            doff0 = half * E2
            dst_r = dsts[r]
            pltpu.sync_copy(z1, acc1.at[pl.ds(a1b, 10000)])

            def fill(g, _):
                onesb[pl.ds(g * 16, 16)] = jnp.ones((16,), jnp.float32)
                return _

            lax.fori_loop(0, G, fill, 0, unroll=8)

            def chunk_body3(c, _):
                pltpu.sync_copy(dst_r.at[pl.ds(doff0 + c * C, C)], didx0)
                pltpu.sync_copy(onesb, acc1.at[pl.ds(a1b, 10000)].at[didx0],
                                add=True)
                return _

            lax.fori_loop(0, E2 // C, chunk_body3, 0)
            pltpu.sync_copy(acc1.at[pl.ds(a1b, 10000)],
                            degp.at[pl.ds((r * 2 + half) * 10000, 10000)])


_sc_aggregate = functools.partial(
    pl.kernel,
    mesh=_sc_mesh,
    compiler_params=_sc_params,
    out_type=[
        jax.ShapeDtypeStruct((4 * 10000, 256), jnp.float32),
        jax.ShapeDtypeStruct((2 * 4 * 10000, 16), jnp.float32),
        jax.ShapeDtypeStruct((8 * 10000,), jnp.float32),
    ],
    scratch_types=[
        pltpu.VMEM_SHARED((NSUB * 10000, 8), jnp.float32),
        pltpu.VMEM_SHARED((4 * 10000,), jnp.float32),
        pltpu.VMEM((C,), jnp.int32),
        pltpu.VMEM((C,), jnp.int32),
        pltpu.VMEM((C,), jnp.int32),
        pltpu.VMEM((C,), jnp.int32),
        pltpu.VMEM((C, 8), jnp.float32),
        pltpu.VMEM((C, 8), jnp.float32),
        pltpu.VMEM((C,), jnp.float32),
        pltpu.SemaphoreType.DMA,
        pltpu.SemaphoreType.DMA,
        pltpu.SemaphoreType.DMA,
        pltpu.SemaphoreType.DMA,
        pltpu.SemaphoreType.DMA,
        pltpu.SemaphoreType.DMA,
    ],
)(_sc_body)


def _p1_body(a0, e0, d0, a1, e1, d1, W0, We0, W1r, We1r, W1, b1, w2r,
             z0o, z1o, s0o, s1o, acc):
    i = pl.program_id(0)

    @pl.when(i == 0)
    def _():
        acc[0] = 0.0
        acc[1] = 0.0

    rd0 = 1.0 / jnp.maximum(d0[...], 1.0)
    rd1 = 1.0 / jnp.maximum(d1[...], 1.0)
    z0 = (jnp.dot(a0[...], W0[...], preferred_element_type=jnp.float32)
          + jnp.dot(e0[...], We0[...], preferred_element_type=jnp.float32)) * rd0
    z1 = (jnp.dot(a1[...], W1r[...], preferred_element_type=jnp.float32)
          + jnp.dot(e1[...], We1r[...], preferred_element_type=jnp.float32)) * rd1
    h0 = jnp.tanh(jnp.dot(z0, W1[...], preferred_element_type=jnp.float32) + b1[...])
    h1 = jnp.tanh(jnp.dot(z1, W1[...], preferred_element_type=jnp.float32) + b1[...])
    acc[0] += jnp.sum(h0 * w2r[...])
    acc[1] += jnp.sum(h1 * w2r[...])
    z0o[...] = z0
    z1o[...] = z1

    @pl.when(i == NBLK - 1)
    def _():
        s0o[...] = jnp.full((1, 128), acc[0], jnp.float32)
        s1o[...] = jnp.full((1, 128), acc[1], jnp.float32)


def _p2_body(z0, z1, s0, s1, o):
    t0 = s0[0, 0] * (1.0 / NU)
    t1 = s1[0, 0] * (1.0 / NU)
    m = jnp.maximum(t0, t1)
    e0 = jnp.exp(t0 - m)
    e1 = jnp.exp(t1 - m)
    b0 = e0 / (e0 + e1)
    b1 = e1 / (e0 + e1)
    o[...] = b0 * z0[...] + b1 * z1[...]


def _dense_pair(a0, e0, d0, a1, e1, d1, W0, We0, W1r, We1r, W1, b1, w2):
    row = lambda i: (i, 0)
    const = lambda i: (0, 0)
    z0, z1, s0, s1 = pl.pallas_call(
        _p1_body,
        grid=(NBLK,),
        in_specs=[
            pl.BlockSpec((NB, D), row),
            pl.BlockSpec((NB, DEA), row),
            pl.BlockSpec((NB, 1), row),
            pl.BlockSpec((NB, D), row),
            pl.BlockSpec((NB, DEA), row),
            pl.BlockSpec((NB, 1), row),
            pl.BlockSpec((D, D), const),
            pl.BlockSpec((DEA, D), const),
            pl.BlockSpec((D, D), const),
            pl.BlockSpec((DEA, D), const),
            pl.BlockSpec((D, HID), const),
            pl.BlockSpec((1, HID), const),
            pl.BlockSpec((1, HID), const),
        ],
        out_specs=[
            pl.BlockSpec((NB, D), row),
            pl.BlockSpec((NB, D), row),
            pl.BlockSpec((1, 128), const),
            pl.BlockSpec((1, 128), const),
        ],
        out_shape=[
            jax.ShapeDtypeStruct((NU, D), jnp.float32),
            jax.ShapeDtypeStruct((NU, D), jnp.float32),
            jax.ShapeDtypeStruct((1, 128), jnp.float32),
            jax.ShapeDtypeStruct((1, 128), jnp.float32),
        ],
        scratch_shapes=[pltpu.SMEM((2,), jnp.float32)],
    )(a0, e0, d0.reshape(NU, 1), a1, e1, d1.reshape(NU, 1),
      W0, We0, W1r, We1r, W1, b1.reshape(1, HID), w2.reshape(1, HID))

    out = pl.pallas_call(
        _p2_body,
        grid=(NBLK,),
        in_specs=[
            pl.BlockSpec((NB, D), row),
            pl.BlockSpec((NB, D), row),
            pl.BlockSpec((1, 128), const),
            pl.BlockSpec((1, 128), const),
        ],
        out_specs=pl.BlockSpec((NB, D), row),
        out_shape=jax.ShapeDtypeStruct((NU, D), jnp.float32),
    )(z0, z1, s0, s1)
    return out.reshape(NU, H, DH)


def kernel(x_user, x_item, ei_follows, ei_boughtby, ei_buys, ei_similar,
           ea_follows, ea_boughtby, ea_buys, ea_similar,
           W_follows, We_follows, W_boughtby, We_boughtby,
           W_buys, We_buys, W_similar, We_similar,
           W1_u, b1_u, w2_u, W1_i, b1_i, w2_i):
    xu3 = x_user.reshape(NU, 32, 8).transpose(1, 0, 2).reshape(-1, 8)
    xi3 = x_item.reshape(NI, 32, 8).transpose(1, 0, 2).reshape(-1, 8)
    xtab = jnp.concatenate([xu3, xi3], axis=0)

    z2 = jnp.zeros((10000, 8), jnp.float32)
    z1 = jnp.zeros((10000,), jnp.float32)

    aggx, aggeP, degp = _sc_aggregate(
        xtab,
        ei_follows[0], ei_follows[1], ei_boughtby[0], ei_boughtby[1],
        ei_buys[0], ei_buys[1], ei_similar[0], ei_similar[1],
        ea_follows, ea_boughtby, ea_buys, ea_similar, z2, z1)

    aggx4 = aggx.reshape(4, NU, D)
    agge4 = aggeP.reshape(2, 4, NU, DEA).sum(axis=0)
    deg4 = degp.reshape(4, 2, NU).sum(axis=1)

    out_user = _dense_pair(aggx4[0], agge4[0], deg4[0],
                           aggx4[1], agge4[1], deg4[1],
                           W_follows, We_follows, W_boughtby, We_boughtby,
                           W1_u, b1_u, w2_u)
    out_item = _dense_pair(aggx4[2], agge4[2], deg4[2],
                           aggx4[3], agge4[3], deg4[3],
                           W_buys, We_buys, W_similar, We_similar,
                           W1_i, b1_i, w2_i)
    return (out_user, out_item)
